# baseline (speedup 1.0000x reference)
import functools

import jax
import jax.numpy as jnp
from jax import lax
from jax.experimental import pallas as pl
from jax.experimental.pallas import tpu as pltpu
from jax.experimental.pallas import tpu_sc as plsc

_NUM_EMB = 8192
_DIM = 64
_PAD = 128
_BETA = 0.25
_TOKENS = 9216
_TB = 128
_NB = _TOKENS // _TB
_NW = 32
_BPW = _TOKENS // _NW
_CHUNK = 96
_NCHUNK = _BPW // _CHUNK


def _tc_body(z_ref, emb_ref, pw_ref, pbr_ref, idx_ref, qcb_ref,
             cbn_scr, iota_scr, zn_scr):
    step = pl.program_id(0)

    @pl.when(step == 0)
    def _project():
        iota_scr[...] = lax.broadcasted_iota(
            jnp.int32, (1, 128), 1).astype(jnp.float32)
        emb = emb_ref[...]
        qcb = lax.dot_general(
            emb, pw_ref[...], (((1,), (1,)), ((), ())),
            preferred_element_type=jnp.float32) + pbr_ref[...]
        qcb_ref[...] = jnp.concatenate(
            [qcb, jnp.zeros((_NUM_EMB, _PAD - _DIM), jnp.float32)], axis=1)
        n = jnp.sqrt(jnp.sum(qcb * qcb, axis=1, keepdims=True))
        cbn_scr[...] = qcb / jnp.maximum(n, 1e-12)
        z = z_ref[...]
        zn_scr[...] = z / jnp.maximum(
            jnp.sqrt(jnp.sum(z * z, axis=1, keepdims=True)), 1e-12)

    zn = zn_scr[pl.ds(step * _TB, _TB), :]
    s = lax.dot_general(
        zn, cbn_scr[...], (((1,), (1,)), ((), ())),
        preferred_element_type=jnp.float32)
    bv = lax.slice(s, (0, 0), (_TB, 128))
    bc = jnp.zeros((_TB, 128), jnp.float32)
    for j in range(1, _NUM_EMB // 128):
        sj = lax.slice(s, (0, j * 128), (_TB, (j + 1) * 128))
        gt = sj > bv
        bv = jnp.maximum(bv, sj)
        bc = jnp.where(gt, jnp.float32(j), bc)
    m = jnp.max(bv, axis=1, keepdims=True)
    eq = bv == m
    big = jnp.float32(_NUM_EMB)
    cmin = jnp.min(jnp.where(eq, bc, big), axis=1, keepdims=True)
    lane = jnp.broadcast_to(iota_scr[...], (_TB, 128))
    eq2 = jnp.logical_and(eq, bc == cmin)
    lmin = jnp.min(jnp.where(eq2, lane, big), axis=1, keepdims=True)
    idx_ref[...] = (cmin * 128 + lmin).astype(jnp.int32)


def _tc_call(zf, emb_weight, proj_w, proj_b2d):
    return pl.pallas_call(
        _tc_body,
        grid=(_NB,),
        in_specs=[
            pl.BlockSpec((_TOKENS, _DIM), lambda i: (0, 0)),
            pl.BlockSpec((_NUM_EMB, _DIM), lambda i: (0, 0)),
            pl.BlockSpec((_DIM, _DIM), lambda i: (0, 0)),
            pl.BlockSpec((1, _DIM), lambda i: (0, 0)),
        ],
        out_specs=[
            pl.BlockSpec((_TB, 1), lambda i: (i, 0)),
            pl.BlockSpec((_NUM_EMB, _PAD), lambda i: (0, 0)),
        ],
        out_shape=[
            jax.ShapeDtypeStruct((_TOKENS, 1), jnp.int32),
            jax.ShapeDtypeStruct((_NUM_EMB, _PAD), jnp.float32),
        ],
        scratch_shapes=[pltpu.VMEM((_NUM_EMB, _DIM), jnp.float32),
                        pltpu.VMEM((1, 128), jnp.float32),
                        pltpu.VMEM((_TOKENS, _DIM), jnp.float32)],
        compiler_params=pltpu.CompilerParams(
            dimension_semantics=("arbitrary",)),
    )(zf, emb_weight, proj_w, proj_b2d)


@functools.lru_cache(maxsize=1)
def _sc_gather_fn():
    mesh = plsc.VectorSubcoreMesh(core_axis_name="c", subcore_axis_name="s")

    @functools.partial(
        pl.kernel,
        mesh=mesh,
        out_type=[
            jax.ShapeDtypeStruct((_TOKENS, _PAD), jnp.float32),
            jax.ShapeDtypeStruct((_NW, 16), jnp.float32),
        ],
        scratch_types=[
            pltpu.VMEM((_BPW,), jnp.int32),
            pltpu.VMEM((_BPW, _PAD), jnp.float32),
            pltpu.VMEM((_BPW * _DIM,), jnp.float32),
            pltpu.VMEM((16,), jnp.float32),
            pltpu.SemaphoreType.DMA,
        ],
    )
    def _sc_gather(qcb_hbm, idx_hbm, zflat_hbm, out_hbm, loss_hbm,
                   idx_v, rows_v, z_v, acc_v, sem):
        c = lax.axis_index("c")
        s = lax.axis_index("s")
        wid = s * 2 + c
        base = wid * _BPW
        pltpu.sync_copy(idx_hbm.at[pl.ds(base, _BPW)], idx_v)
        for j in range(_NCHUNK):
            pltpu.async_copy(
                qcb_hbm.at[idx_v.at[pl.ds(j * _CHUNK, _CHUNK)]],
                rows_v.at[pl.ds(j * _CHUNK, _CHUNK)], sem)
        pltpu.sync_copy(zflat_hbm.at[pl.ds(base * _DIM, _BPW * _DIM)], z_v)
        for j in range(_NCHUNK):
            pltpu.make_async_copy(
                qcb_hbm.at[idx_v.at[pl.ds(j * _CHUNK, _CHUNK)]],
                rows_v.at[pl.ds(j * _CHUNK, _CHUNK)], sem).wait()

        def body(i, acc):
            for k in range(_DIM // 16):
                q = rows_v[i, pl.ds(k * 16, 16)]
                zz = z_v[pl.ds(i * _DIM + k * 16, 16)]
                d = q - zz
                acc = acc + d * d
            return acc

        acc = lax.fori_loop(0, _BPW, body, jnp.zeros((16,), jnp.float32))
        acc_v[...] = acc
        pltpu.sync_copy(rows_v, out_hbm.at[pl.ds(base, _BPW)])
        pltpu.sync_copy(acc_v, loss_hbm.at[wid])

    return _sc_gather


def kernel(z, emb_weight, proj_w, proj_b, l2_scale):
    del l2_scale
    B, T, D = z.shape
    zf = z.reshape(-1, D)
    idx2d, qcb_pad = _tc_call(zf, emb_weight, proj_w, proj_b.reshape(1, D))
    idx = idx2d.reshape(-1)
    quant_pad, loss_rows = _sc_gather_fn()(qcb_pad, idx, zf.reshape(-1))
    quant = quant_pad[:, :_DIM]
    vq_loss = (1.0 + _BETA) * jnp.sum(loss_rows) / zf.size
    return quant.reshape(z.shape), vq_loss, idx.reshape(B, T)


# speedup vs baseline: 1.0074x; 1.0074x over previous
import functools

import jax
import jax.numpy as jnp
from jax import lax
from jax.experimental import pallas as pl
from jax.experimental.pallas import tpu as pltpu
from jax.experimental.pallas import tpu_sc as plsc

_NUM_EMB = 8192
_DIM = 64
_PAD = 128
_BETA = 0.25
_TOKENS = 9216
_TB = 128
_NB = _TOKENS // _TB
_NW = 32
_BPW = _TOKENS // _NW
_CHUNK = 96
_NCHUNK = _BPW // _CHUNK


def _tc_body(z_ref, emb_ref, pw_ref, pbr_ref, idx_ref, qcb_ref,
             cbn_scr, iota_scr, zn_scr):
    step = pl.program_id(0)

    @pl.when(step == 0)
    def _project():
        iota_scr[...] = lax.broadcasted_iota(
            jnp.int32, (1, 128), 1).astype(jnp.float32)
        emb = emb_ref[...]
        qcb = lax.dot_general(
            emb, pw_ref[...], (((1,), (1,)), ((), ())),
            preferred_element_type=jnp.float32) + pbr_ref[...]
        qcb_ref[...] = jnp.concatenate(
            [qcb, jnp.zeros((_NUM_EMB, _PAD - _DIM), jnp.float32)], axis=1)
        n = jnp.sqrt(jnp.sum(qcb * qcb, axis=1, keepdims=True))
        cbn_scr[...] = qcb / jnp.maximum(n, 1e-12)
        z = z_ref[...]
        zn_scr[...] = z / jnp.maximum(
            jnp.sqrt(jnp.sum(z * z, axis=1, keepdims=True)), 1e-12)

    zn = zn_scr[pl.ds(step * _TB, _TB), :]
    s = lax.dot_general(
        zn, cbn_scr[...], (((1,), (1,)), ((), ())),
        preferred_element_type=jnp.float32)
    bv = lax.slice(s, (0, 0), (_TB, 128))
    bc = jnp.zeros((_TB, 128), jnp.float32)
    for j in range(1, _NUM_EMB // 128):
        sj = lax.slice(s, (0, j * 128), (_TB, (j + 1) * 128))
        gt = sj > bv
        bv = jnp.maximum(bv, sj)
        bc = jnp.where(gt, jnp.float32(j), bc)
    m = jnp.max(bv, axis=1, keepdims=True)
    eq = bv == m
    big = jnp.float32(_NUM_EMB)
    cmin = jnp.min(jnp.where(eq, bc, big), axis=1, keepdims=True)
    lane = jnp.broadcast_to(iota_scr[...], (_TB, 128))
    eq2 = jnp.logical_and(eq, bc == cmin)
    lmin = jnp.min(jnp.where(eq2, lane, big), axis=1, keepdims=True)
    idx_ref[...] = (cmin * 128 + lmin).astype(jnp.int32)


def _tc_call(zf, emb_weight, proj_w, proj_b2d):
    return pl.pallas_call(
        _tc_body,
        grid=(_NB,),
        in_specs=[
            pl.BlockSpec((_TOKENS, _DIM), lambda i: (0, 0)),
            pl.BlockSpec((_NUM_EMB, _DIM), lambda i: (0, 0)),
            pl.BlockSpec((_DIM, _DIM), lambda i: (0, 0)),
            pl.BlockSpec((1, _DIM), lambda i: (0, 0)),
        ],
        out_specs=[
            pl.BlockSpec((_TB, 1), lambda i: (i, 0)),
            pl.BlockSpec((_NUM_EMB, _PAD), lambda i: (0, 0)),
        ],
        out_shape=[
            jax.ShapeDtypeStruct((_TOKENS, 1), jnp.int32),
            jax.ShapeDtypeStruct((_NUM_EMB, _PAD), jnp.float32),
        ],
        scratch_shapes=[pltpu.VMEM((_NUM_EMB, _DIM), jnp.float32),
                        pltpu.VMEM((1, 128), jnp.float32),
                        pltpu.VMEM((_TOKENS, _DIM), jnp.float32)],
        compiler_params=pltpu.CompilerParams(
            dimension_semantics=("arbitrary",)),
    )(zf, emb_weight, proj_w, proj_b2d)


@functools.lru_cache(maxsize=1)
def _sc_gather_fn():
    mesh = plsc.VectorSubcoreMesh(core_axis_name="c", subcore_axis_name="s")

    @functools.partial(
        pl.kernel,
        mesh=mesh,
        out_type=[
            jax.ShapeDtypeStruct((_TOKENS, _PAD), jnp.float32),
            jax.ShapeDtypeStruct((_NW, 16), jnp.float32),
        ],
        scratch_types=[
            pltpu.VMEM((_BPW,), jnp.int32),
            pltpu.VMEM((_BPW, _PAD), jnp.float32),
            pltpu.VMEM((_BPW * _DIM,), jnp.float32),
            pltpu.VMEM((16,), jnp.float32),
            pltpu.SemaphoreType.DMA,
        ],
    )
    def _sc_gather(qcb_hbm, idx_hbm, zflat_hbm, out_hbm, loss_hbm,
                   idx_v, rows_v, z_v, acc_v, sem):
        c = lax.axis_index("c")
        s = lax.axis_index("s")
        wid = s * 2 + c
        base = wid * _BPW
        pltpu.sync_copy(idx_hbm.at[pl.ds(base, _BPW)], idx_v)
        for j in range(_NCHUNK):
            pltpu.async_copy(
                qcb_hbm.at[idx_v.at[pl.ds(j * _CHUNK, _CHUNK)]],
                rows_v.at[pl.ds(j * _CHUNK, _CHUNK)], sem)
        pltpu.sync_copy(zflat_hbm.at[pl.ds(base * _DIM, _BPW * _DIM)], z_v)
        for j in range(_NCHUNK):
            pltpu.make_async_copy(
                qcb_hbm.at[idx_v.at[pl.ds(j * _CHUNK, _CHUNK)]],
                rows_v.at[pl.ds(j * _CHUNK, _CHUNK)], sem).wait()
        out_cp = pltpu.make_async_copy(
            rows_v, out_hbm.at[pl.ds(base, _BPW)], sem)
        out_cp.start()

        def body(i, acc):
            for k in range(_DIM // 16):
                q = rows_v[i, pl.ds(k * 16, 16)]
                zz = z_v[pl.ds(i * _DIM + k * 16, 16)]
                d = q - zz
                acc = acc + d * d
            return acc

        acc = lax.fori_loop(0, _BPW, body, jnp.zeros((16,), jnp.float32))
        acc_v[...] = acc
        out_cp.wait()
        pltpu.sync_copy(acc_v, loss_hbm.at[wid])

    return _sc_gather


def kernel(z, emb_weight, proj_w, proj_b, l2_scale):
    del l2_scale
    B, T, D = z.shape
    zf = z.reshape(-1, D)
    idx2d, qcb_pad = _tc_call(zf, emb_weight, proj_w, proj_b.reshape(1, D))
    idx = idx2d.reshape(-1)
    quant_pad, loss_rows = _sc_gather_fn()(qcb_pad, idx, zf.reshape(-1))
    quant = quant_pad[:, :_DIM]
    vq_loss = (1.0 + _BETA) * jnp.sum(loss_rows) / zf.size
    return quant.reshape(z.shape), vq_loss, idx.reshape(B, T)


# speedup vs baseline: 1.1718x; 1.1632x over previous
import functools

import jax
import jax.numpy as jnp
from jax import lax
from jax.experimental import pallas as pl
from jax.experimental.pallas import tpu as pltpu
from jax.experimental.pallas import tpu_sc as plsc

_NUM_EMB = 8192
_DIM = 64
_PAD = 128
_BETA = 0.25
_TOKENS = 9216
_TB = 256
_NB = _TOKENS // _TB
_NW = 32
_BPW = _TOKENS // _NW
_CHUNK = 96
_NCHUNK = _BPW // _CHUNK


def _tc_body(z_ref, emb_ref, pw_ref, pbr_ref, idx_ref, qcb_ref,
             cbn_scr, iota_scr, zn_scr):
    step = pl.program_id(0)

    @pl.when(step == 0)
    def _project():
        iota_scr[...] = lax.broadcasted_iota(
            jnp.int32, (1, 128), 1).astype(jnp.float32)
        emb = emb_ref[...]
        qcb = lax.dot_general(
            emb, pw_ref[...], (((1,), (1,)), ((), ())),
            preferred_element_type=jnp.float32) + pbr_ref[...]
        qcb_ref[...] = jnp.concatenate(
            [qcb, jnp.zeros((_NUM_EMB, _PAD - _DIM), jnp.float32)], axis=1)
        n = jnp.sqrt(jnp.sum(qcb * qcb, axis=1, keepdims=True))
        cbn_scr[...] = qcb / jnp.maximum(n, 1e-12)
        z = z_ref[...]
        zn_scr[...] = z / jnp.maximum(
            jnp.sqrt(jnp.sum(z * z, axis=1, keepdims=True)), 1e-12)

    zn = zn_scr[pl.ds(step * _TB, _TB), :]
    s = lax.dot_general(
        zn, cbn_scr[...], (((1,), (1,)), ((), ())),
        preferred_element_type=jnp.float32)
    bv = lax.slice(s, (0, 0), (_TB, 128))
    bc = jnp.zeros((_TB, 128), jnp.float32)
    for j in range(1, _NUM_EMB // 128):
        sj = lax.slice(s, (0, j * 128), (_TB, (j + 1) * 128))
        gt = sj > bv
        bv = jnp.maximum(bv, sj)
        bc = jnp.where(gt, jnp.float32(j), bc)
    m = jnp.max(bv, axis=1, keepdims=True)
    eq = bv == m
    big = jnp.float32(_NUM_EMB)
    cmin = jnp.min(jnp.where(eq, bc, big), axis=1, keepdims=True)
    lane = jnp.broadcast_to(iota_scr[...], (_TB, 128))
    eq2 = jnp.logical_and(eq, bc == cmin)
    lmin = jnp.min(jnp.where(eq2, lane, big), axis=1, keepdims=True)
    idx_ref[...] = (cmin * 128 + lmin).astype(jnp.int32)


def _tc_call(zf, emb_weight, proj_w, proj_b2d):
    return pl.pallas_call(
        _tc_body,
        grid=(_NB,),
        in_specs=[
            pl.BlockSpec((_TOKENS, _DIM), lambda i: (0, 0)),
            pl.BlockSpec((_NUM_EMB, _DIM), lambda i: (0, 0)),
            pl.BlockSpec((_DIM, _DIM), lambda i: (0, 0)),
            pl.BlockSpec((1, _DIM), lambda i: (0, 0)),
        ],
        out_specs=[
            pl.BlockSpec((_TB, 1), lambda i: (i, 0)),
            pl.BlockSpec((_NUM_EMB, _PAD), lambda i: (0, 0)),
        ],
        out_shape=[
            jax.ShapeDtypeStruct((_TOKENS, 1), jnp.int32),
            jax.ShapeDtypeStruct((_NUM_EMB, _PAD), jnp.float32),
        ],
        scratch_shapes=[pltpu.VMEM((_NUM_EMB, _DIM), jnp.float32),
                        pltpu.VMEM((1, 128), jnp.float32),
                        pltpu.VMEM((_TOKENS, _DIM), jnp.float32)],
        compiler_params=pltpu.CompilerParams(
            dimension_semantics=("arbitrary",)),
    )(zf, emb_weight, proj_w, proj_b2d)


@functools.lru_cache(maxsize=1)
def _sc_gather_fn():
    mesh = plsc.VectorSubcoreMesh(core_axis_name="c", subcore_axis_name="s")

    @functools.partial(
        pl.kernel,
        mesh=mesh,
        out_type=[
            jax.ShapeDtypeStruct((_TOKENS, _PAD), jnp.float32),
            jax.ShapeDtypeStruct((_NW, 16), jnp.float32),
        ],
        scratch_types=[
            pltpu.VMEM((_BPW,), jnp.int32),
            pltpu.VMEM((_BPW, _PAD), jnp.float32),
            pltpu.VMEM((_BPW * _DIM,), jnp.float32),
            pltpu.VMEM((16,), jnp.float32),
            pltpu.SemaphoreType.DMA,
        ],
    )
    def _sc_gather(qcb_hbm, idx_hbm, zflat_hbm, out_hbm, loss_hbm,
                   idx_v, rows_v, z_v, acc_v, sem):
        c = lax.axis_index("c")
        s = lax.axis_index("s")
        wid = s * 2 + c
        base = wid * _BPW
        pltpu.sync_copy(idx_hbm.at[pl.ds(base, _BPW)], idx_v)
        for j in range(_NCHUNK):
            pltpu.async_copy(
                qcb_hbm.at[idx_v.at[pl.ds(j * _CHUNK, _CHUNK)]],
                rows_v.at[pl.ds(j * _CHUNK, _CHUNK)], sem)
        pltpu.sync_copy(zflat_hbm.at[pl.ds(base * _DIM, _BPW * _DIM)], z_v)
        for j in range(_NCHUNK):
            pltpu.make_async_copy(
                qcb_hbm.at[idx_v.at[pl.ds(j * _CHUNK, _CHUNK)]],
                rows_v.at[pl.ds(j * _CHUNK, _CHUNK)], sem).wait()
        out_cp = pltpu.make_async_copy(
            rows_v, out_hbm.at[pl.ds(base, _BPW)], sem)
        out_cp.start()

        def body(i, acc):
            for k in range(_DIM // 16):
                q = rows_v[i, pl.ds(k * 16, 16)]
                zz = z_v[pl.ds(i * _DIM + k * 16, 16)]
                d = q - zz
                acc = acc + d * d
            return acc

        acc = lax.fori_loop(0, _BPW, body, jnp.zeros((16,), jnp.float32))
        acc_v[...] = acc
        out_cp.wait()
        pltpu.sync_copy(acc_v, loss_hbm.at[wid])

    return _sc_gather


def kernel(z, emb_weight, proj_w, proj_b, l2_scale):
    del l2_scale
    B, T, D = z.shape
    zf = z.reshape(-1, D)
    idx2d, qcb_pad = _tc_call(zf, emb_weight, proj_w, proj_b.reshape(1, D))
    idx = idx2d.reshape(-1)
    quant_pad, loss_rows = _sc_gather_fn()(qcb_pad, idx, zf.reshape(-1))
    quant = quant_pad[:, :_DIM]
    vq_loss = (1.0 + _BETA) * jnp.sum(loss_rows) / zf.size
    return quant.reshape(z.shape), vq_loss, idx.reshape(B, T)


# speedup vs baseline: 1.2649x; 1.0794x over previous
import functools

import jax
import jax.numpy as jnp
from jax import lax
from jax.experimental import pallas as pl
from jax.experimental.pallas import tpu as pltpu
from jax.experimental.pallas import tpu_sc as plsc

_NUM_EMB = 8192
_DIM = 64
_PAD = 128
_BETA = 0.25
_TOKENS = 9216
_TB = 512
_NB = _TOKENS // _TB
_NW = 32
_BPW = _TOKENS // _NW
_CHUNK = 96
_NCHUNK = _BPW // _CHUNK


def _tc_body(z_ref, emb_ref, pw_ref, pbr_ref, idx_ref, qcb_ref,
             cbn_scr, iota_scr, zn_scr):
    step = pl.program_id(0)

    @pl.when(step == 0)
    def _project():
        iota_scr[...] = lax.broadcasted_iota(
            jnp.int32, (1, 128), 1).astype(jnp.float32)
        emb = emb_ref[...]
        qcb = lax.dot_general(
            emb, pw_ref[...], (((1,), (1,)), ((), ())),
            preferred_element_type=jnp.float32) + pbr_ref[...]
        qcb_ref[...] = jnp.concatenate(
            [qcb, jnp.zeros((_NUM_EMB, _PAD - _DIM), jnp.float32)], axis=1)
        n = jnp.sqrt(jnp.sum(qcb * qcb, axis=1, keepdims=True))
        cbn_scr[...] = qcb / jnp.maximum(n, 1e-12)
        z = z_ref[...]
        zn_scr[...] = z / jnp.maximum(
            jnp.sqrt(jnp.sum(z * z, axis=1, keepdims=True)), 1e-12)

    zn = zn_scr[pl.ds(step * _TB, _TB), :]
    s = lax.dot_general(
        zn, cbn_scr[...], (((1,), (1,)), ((), ())),
        preferred_element_type=jnp.float32)
    bv = lax.slice(s, (0, 0), (_TB, 128))
    bc = jnp.zeros((_TB, 128), jnp.float32)
    for j in range(1, _NUM_EMB // 128):
        sj = lax.slice(s, (0, j * 128), (_TB, (j + 1) * 128))
        gt = sj > bv
        bv = jnp.maximum(bv, sj)
        bc = jnp.where(gt, jnp.float32(j), bc)
    m = jnp.max(bv, axis=1, keepdims=True)
    eq = bv == m
    big = jnp.float32(_NUM_EMB)
    cmin = jnp.min(jnp.where(eq, bc, big), axis=1, keepdims=True)
    lane = jnp.broadcast_to(iota_scr[...], (_TB, 128))
    eq2 = jnp.logical_and(eq, bc == cmin)
    lmin = jnp.min(jnp.where(eq2, lane, big), axis=1, keepdims=True)
    idx_ref[...] = (cmin * 128 + lmin).astype(jnp.int32)


def _tc_call(zf, emb_weight, proj_w, proj_b2d):
    return pl.pallas_call(
        _tc_body,
        grid=(_NB,),
        in_specs=[
            pl.BlockSpec((_TOKENS, _DIM), lambda i: (0, 0)),
            pl.BlockSpec((_NUM_EMB, _DIM), lambda i: (0, 0)),
            pl.BlockSpec((_DIM, _DIM), lambda i: (0, 0)),
            pl.BlockSpec((1, _DIM), lambda i: (0, 0)),
        ],
        out_specs=[
            pl.BlockSpec((_TB, 1), lambda i: (i, 0)),
            pl.BlockSpec((_NUM_EMB, _PAD), lambda i: (0, 0)),
        ],
        out_shape=[
            jax.ShapeDtypeStruct((_TOKENS, 1), jnp.int32),
            jax.ShapeDtypeStruct((_NUM_EMB, _PAD), jnp.float32),
        ],
        scratch_shapes=[pltpu.VMEM((_NUM_EMB, _DIM), jnp.float32),
                        pltpu.VMEM((1, 128), jnp.float32),
                        pltpu.VMEM((_TOKENS, _DIM), jnp.float32)],
        compiler_params=pltpu.CompilerParams(
            dimension_semantics=("arbitrary",)),
    )(zf, emb_weight, proj_w, proj_b2d)


@functools.lru_cache(maxsize=1)
def _sc_gather_fn():
    mesh = plsc.VectorSubcoreMesh(core_axis_name="c", subcore_axis_name="s")

    @functools.partial(
        pl.kernel,
        mesh=mesh,
        out_type=[
            jax.ShapeDtypeStruct((_TOKENS, _PAD), jnp.float32),
            jax.ShapeDtypeStruct((_NW, 16), jnp.float32),
        ],
        scratch_types=[
            pltpu.VMEM((_BPW,), jnp.int32),
            pltpu.VMEM((_BPW, _PAD), jnp.float32),
            pltpu.VMEM((_BPW * _DIM,), jnp.float32),
            pltpu.VMEM((16,), jnp.float32),
            pltpu.SemaphoreType.DMA,
        ],
    )
    def _sc_gather(qcb_hbm, idx_hbm, zflat_hbm, out_hbm, loss_hbm,
                   idx_v, rows_v, z_v, acc_v, sem):
        c = lax.axis_index("c")
        s = lax.axis_index("s")
        wid = s * 2 + c
        base = wid * _BPW
        pltpu.sync_copy(idx_hbm.at[pl.ds(base, _BPW)], idx_v)
        for j in range(_NCHUNK):
            pltpu.async_copy(
                qcb_hbm.at[idx_v.at[pl.ds(j * _CHUNK, _CHUNK)]],
                rows_v.at[pl.ds(j * _CHUNK, _CHUNK)], sem)
        pltpu.sync_copy(zflat_hbm.at[pl.ds(base * _DIM, _BPW * _DIM)], z_v)
        for j in range(_NCHUNK):
            pltpu.make_async_copy(
                qcb_hbm.at[idx_v.at[pl.ds(j * _CHUNK, _CHUNK)]],
                rows_v.at[pl.ds(j * _CHUNK, _CHUNK)], sem).wait()
        out_cp = pltpu.make_async_copy(
            rows_v, out_hbm.at[pl.ds(base, _BPW)], sem)
        out_cp.start()

        def body(i, acc):
            for k in range(_DIM // 16):
                q = rows_v[i, pl.ds(k * 16, 16)]
                zz = z_v[pl.ds(i * _DIM + k * 16, 16)]
                d = q - zz
                acc = acc + d * d
            return acc

        acc = lax.fori_loop(0, _BPW, body, jnp.zeros((16,), jnp.float32))
        acc_v[...] = acc
        out_cp.wait()
        pltpu.sync_copy(acc_v, loss_hbm.at[wid])

    return _sc_gather


def kernel(z, emb_weight, proj_w, proj_b, l2_scale):
    del l2_scale
    B, T, D = z.shape
    zf = z.reshape(-1, D)
    idx2d, qcb_pad = _tc_call(zf, emb_weight, proj_w, proj_b.reshape(1, D))
    idx = idx2d.reshape(-1)
    quant_pad, loss_rows = _sc_gather_fn()(qcb_pad, idx, zf.reshape(-1))
    quant = quant_pad[:, :_DIM]
    vq_loss = (1.0 + _BETA) * jnp.sum(loss_rows) / zf.size
    return quant.reshape(z.shape), vq_loss, idx.reshape(B, T)
